# baseline (speedup 1.0000x reference)
import functools

import jax
import jax.numpy as jnp
from jax import lax
from jax.experimental import pallas as pl
from jax.experimental.pallas import tpu as pltpu
from jax.experimental.pallas import tpu_sc as plsc

_RATIO1, _RATIO2 = 0.5, 0.25
_R1, _R2 = 0.2, 0.4
_MAX_NB = 64
_N, _B = 8192, 8

_NEG = float("-inf")


def _r2_f32(r):
    import numpy as np
    t = float(r) * float(r)
    t32 = np.float32(t)
    if float(t32) > t:
        t32 = np.nextafter(t32, np.float32(0.0))
    return t32


def _fps_body(mmax_ref, px_ref, py_ref, pz_ref, vm_ref, out_ref, dist_ref):
    B, P = px_ref.shape
    px = px_ref[...]
    py = py_ref[...]
    pz = pz_ref[...]
    d0 = ((px - px[:, 0:1]) ** 2 + (py - py[:, 0:1]) ** 2
          + (pz - pz[:, 0:1]) ** 2)
    dist_ref[...] = jnp.where(vm_ref[...] > 0, d0, _NEG)
    jidx = lax.broadcasted_iota(jnp.int32, (B, P), 1)
    lane = lax.broadcasted_iota(jnp.int32, (B, 128), 1)
    mmax = mmax_ref[0]

    def it(i, buf):
        dist = dist_ref[...]
        maxv = jnp.max(dist, axis=1, keepdims=True)
        sel = dist >= maxv
        jsel = jnp.min(jnp.where(sel, jidx, P), axis=1, keepdims=True)
        onehot = jidx == jsel
        qx = jnp.max(jnp.where(onehot, px, _NEG), axis=1, keepdims=True)
        qy = jnp.max(jnp.where(onehot, py, _NEG), axis=1, keepdims=True)
        qz = jnp.max(jnp.where(onehot, pz, _NEG), axis=1, keepdims=True)
        d2 = (px - qx) ** 2 + (py - qy) ** 2 + (pz - qz) ** 2
        dist_ref[...] = jnp.minimum(dist, d2)
        im = lax.rem(i, 128)
        buf = jnp.where(lane == im, jsel, buf)

        @pl.when(im == 127)
        def _flush():
            base = pl.multiple_of(i - 127, 128)
            out_ref[:, pl.ds(base, 128)] = buf

        return buf

    buf0 = jnp.zeros((B, 128), jnp.int32)
    buf = lax.fori_loop(1, mmax, it, buf0)
    last_base = pl.multiple_of(((mmax - 1) // 128) * 128, 128)
    out_ref[:, pl.ds(last_base, 128)] = buf


def _fps_pallas(px, py, pz, vmask, mmax, mcap, interpret=False):
    B, P = px.shape
    return pl.pallas_call(
        _fps_body,
        out_shape=jax.ShapeDtypeStruct((B, mcap), jnp.int32),
        in_specs=[
            pl.BlockSpec(memory_space=pltpu.SMEM),
            pl.BlockSpec(memory_space=pltpu.VMEM),
            pl.BlockSpec(memory_space=pltpu.VMEM),
            pl.BlockSpec(memory_space=pltpu.VMEM),
            pl.BlockSpec(memory_space=pltpu.VMEM),
        ],
        out_specs=pl.BlockSpec(memory_space=pltpu.VMEM),
        scratch_shapes=[pltpu.VMEM((B, P), jnp.float32)],
        interpret=interpret,
    )(jnp.reshape(mmax.astype(jnp.int32), (1,)), px, py, pz, vmask)


def _fps_level(pos_all, starts, counts, ratio, cap, ppad, mcap, interpret=False):
    P0 = pos_all.shape[0]
    mvec = jnp.maximum(1, jnp.ceil(ratio * counts.astype(jnp.float32)).astype(counts.dtype))
    offs = jnp.concatenate([jnp.zeros((1,), mvec.dtype), jnp.cumsum(mvec)])
    total = offs[-1]
    mmax = jnp.max(mvec)
    maxcnt = jnp.max(counts)

    def run(ppad_x):
        j = jnp.arange(ppad_x)
        gidx = jnp.minimum(starts[:, None] + j[None, :], P0 - 1)
        px = pos_all[:, 0][gidx]
        py = pos_all[:, 1][gidx]
        pz = pos_all[:, 2][gidx]
        vmask = (j[None, :] < counts[:, None]).astype(jnp.float32)
        return _fps_pallas(px, py, pz, vmask, mmax, mcap, interpret=interpret)

    small = ((max(ppad // 4, 256) + 127) // 128) * 128
    jsel = lax.cond(maxcnt <= small,
                    functools.partial(run, small),
                    functools.partial(run, ppad))

    t = jnp.arange(cap)
    b = jnp.clip(jnp.searchsorted(offs, t, side="right") - 1, 0, _B - 1)
    i = jnp.clip(t - offs[b], 0, mcap - 1)
    jsel_t = jsel[b, i]
    gsel = jnp.minimum(starts[b] + jsel_t, P0 - 1)
    valid = t < total
    idx = jnp.where(valid, gsel, 0)
    bid = jnp.where(valid, b, 0).astype(starts.dtype)
    return idx, bid, valid, mvec, offs, jsel


_NBW = 80


def _radius_sc_call(qx, qy, qz, qs, qc, tx, ty, tz, thr, mpad, tlen):
    info = plsc.get_sparse_core_info()
    NW = info.num_cores * info.num_subcores
    L = info.num_lanes
    qpw = mpad // NW
    mesh = plsc.VectorSubcoreMesh(core_axis_name="c", subcore_axis_name="s")

    @functools.partial(
        pl.kernel, mesh=mesh,
        out_type=jax.ShapeDtypeStruct((mpad * _NBW,), jnp.int32),
        compiler_params=pltpu.CompilerParams(needs_layout_passes=False),
        scratch_types=[
            pltpu.VMEM((tlen,), jnp.float32),
            pltpu.VMEM((tlen,), jnp.float32),
            pltpu.VMEM((tlen,), jnp.float32),
            pltpu.VMEM((qpw,), jnp.float32),
            pltpu.VMEM((qpw,), jnp.float32),
            pltpu.VMEM((qpw,), jnp.float32),
            pltpu.VMEM((qpw,), jnp.int32),
            pltpu.VMEM((qpw,), jnp.int32),
            pltpu.VMEM((qpw * _NBW,), jnp.int32),
        ],
    )
    def k(qx_h, qy_h, qz_h, qs_h, qc_h, tx_h, ty_h, tz_h, out_h,
          txv, tyv, tzv, qxv, qyv, qzv, qsv, qcv, nbrv):
        wid = lax.axis_index("s") * info.num_cores + lax.axis_index("c")
        base = wid * qpw
        pltpu.sync_copy(tx_h, txv)
        pltpu.sync_copy(ty_h, tyv)
        pltpu.sync_copy(tz_h, tzv)
        pltpu.sync_copy(qx_h.at[pl.ds(base, qpw)], qxv)
        pltpu.sync_copy(qy_h.at[pl.ds(base, qpw)], qyv)
        pltpu.sync_copy(qz_h.at[pl.ds(base, qpw)], qzv)
        pltpu.sync_copy(qs_h.at[pl.ds(base, qpw)], qsv)
        pltpu.sync_copy(qc_h.at[pl.ds(base, qpw)], qcv)
        lanes = lax.broadcasted_iota(jnp.int32, (L,), 0)
        neg1 = jnp.full((L,), -1, jnp.int32)

        def per_group(g, carry):
            qx16 = qxv[pl.ds(g * L, L)]
            qy16 = qyv[pl.ds(g * L, L)]
            qz16 = qzv[pl.ds(g * L, L)]
            qs16 = qsv[pl.ds(g * L, L)]
            qc16 = qcv[pl.ds(g * L, L)]
            for lq in range(L):
                qxs = qx16[lq]
                qys = qy16[lq]
                qzs = qz16[lq]
                s0 = qs16[lq]
                cnt = qc16[lq]
                rowbase = (g * L + lq) * _NBW
                for s in range(4):
                    nbrv[pl.ds(rowbase + s * L, L)] = neg1
                c0 = s0 // L
                c1 = (s0 + cnt + L - 1) // L

                def body(c, off, s0=s0, cnt=cnt, qxs=qxs, qys=qys, qzs=qzs,
                         rowbase=rowbase):
                    absj = lanes + c * L
                    x = txv[pl.ds(c * L, L)]
                    y = tyv[pl.ds(c * L, L)]
                    z = tzv[pl.ds(c * L, L)]
                    dx = x - qxs
                    dy = y - qys
                    dz = z - qzs
                    d2 = dx * dx + dy * dy + dz * dz
                    mi = ((absj >= s0) & (absj < s0 + cnt)
                          & (d2 <= thr)).astype(jnp.int32)
                    rank = lax.cumsum(mi) - mi
                    mst = (mi > 0) & (off + rank < _MAX_NB)
                    offc = jnp.minimum(off, _MAX_NB)
                    plsc.store_compressed(nbrv.at[pl.ds(rowbase + offc, L)],
                                          absj, mask=mst)
                    return off + jnp.sum(mi)

                lax.fori_loop(c0, c1, body, jnp.int32(0))
            return carry

        lax.fori_loop(0, qpw // L, per_group, 0)
        pltpu.sync_copy(nbrv, out_h.at[pl.ds(base * _NBW, qpw * _NBW)])

    flat = k(qx, qy, qz, qs, qc, tx, ty, tz)
    return flat.reshape(mpad, _NBW)


def _radius_sc(pos_all, starts, counts, q_idx, q_bid, q_val, thr, max_nb,
               mpad, tlen):
    M = q_idx.shape[0]
    P0 = pos_all.shape[0]
    qpos = pos_all[q_idx]
    qs = starts[q_bid].astype(jnp.int32)
    qc = jnp.where(q_val, counts[q_bid], 0).astype(jnp.int32)
    pad = mpad - M
    qx = jnp.pad(qpos[:, 0], (0, pad))
    qy = jnp.pad(qpos[:, 1], (0, pad))
    qz = jnp.pad(qpos[:, 2], (0, pad))
    qs = jnp.pad(qs, (0, pad))
    qc = jnp.pad(qc, (0, pad))
    tpad = tlen - P0
    tx = jnp.pad(pos_all[:, 0], (0, tpad))
    ty = jnp.pad(pos_all[:, 1], (0, tpad))
    tz = jnp.pad(pos_all[:, 2], (0, tpad))
    nbr_raw = _radius_sc_call(qx, qy, qz, qs, qc, tx, ty, tz,
                              float(thr), mpad, tlen)[:M, :max_nb]
    msk = nbr_raw >= 0
    nbr = jnp.where(msk, nbr_raw, 0)
    return nbr, msk


def _radius_xla(pos_all, starts, counts, q_idx, q_bid, q_val, thr, max_nb):
    P = pos_all.shape[0]
    j = jnp.arange(P)

    def one(qi, qb, qv):
        g = jnp.minimum(starts[qb] + j, P - 1)
        d2 = jnp.sum((pos_all[g] - pos_all[qi]) ** 2, axis=-1)
        cond = (j < counts[qb]) & (d2 <= thr) & qv
        rank = jnp.cumsum(cond.astype(jnp.int32)) - 1
        slot = jnp.where(cond & (rank < max_nb), rank, max_nb)
        nbr = jnp.zeros((max_nb + 1,), g.dtype).at[slot].set(g)[:max_nb]
        cnt = jnp.minimum(jnp.sum(cond.astype(jnp.int32)), max_nb)
        msk = jnp.arange(max_nb) < cnt
        return nbr, msk

    return jax.vmap(one)(q_idx, q_bid, q_val)


def _sc_gather_call(table, idx, C):
    E = idx.shape[0]
    T = table.shape[0]
    info = plsc.get_sparse_core_info()
    NW = info.num_cores * info.num_subcores
    epw = E // NW
    nch = epw // 128
    mesh = plsc.VectorSubcoreMesh(core_axis_name="c", subcore_axis_name="s")

    @functools.partial(
        pl.kernel, mesh=mesh,
        out_type=jax.ShapeDtypeStruct((E, C), jnp.float32),
        compiler_params=pltpu.CompilerParams(needs_layout_passes=False),
        scratch_types=[
            pltpu.VMEM((128,), jnp.int32),
            pltpu.VMEM((128, C), jnp.float32),
            pltpu.SemaphoreType.DMA,
        ],
    )
    def k(table_h, idx_h, out_h, idx_v, rows_v, sem):
        wid = lax.axis_index("s") * info.num_cores + lax.axis_index("c")
        base = wid * epw

        def chunk(c, carry):
            off = base + c * 128
            pltpu.sync_copy(idx_h.at[pl.ds(off, 128)], idx_v)
            pltpu.async_copy(table_h.at[idx_v], rows_v, sem).wait()
            pltpu.sync_copy(rows_v, out_h.at[pl.ds(off, 128)])
            return carry

        lax.fori_loop(0, nch, chunk, 0)

    return k(table, idx)


def _sc_gather(table, idx):
    return _sc_gather_call(table, idx, table.shape[1])


def _mm_body(x_ref, w_ref, o_ref):
    o_ref[...] = jnp.dot(x_ref[...], w_ref[...],
                         preferred_element_type=jnp.float32, precision=lax.Precision.HIGHEST)


def _mm(x, w):
    n, cin = x.shape
    cout = w.shape[1]
    return pl.pallas_call(
        _mm_body,
        out_shape=jax.ShapeDtypeStruct((n, cout), jnp.float32),
    )(x, w)


def _edge1_body(g_ref, z_ref, me_ref, b_ref, h_ref, ssum_ref, ssq_ref,
                cnt_ref, *, bq, nb, cg):
    i = pl.program_id(0)

    @pl.when(i == 0)
    def _init():
        ssum_ref[...] = jnp.zeros_like(ssum_ref)
        ssq_ref[...] = jnp.zeros_like(ssq_ref)
        cnt_ref[...] = jnp.zeros_like(cnt_ref)

    g = g_ref[...][:, :cg]
    z = z_ref[...]
    ze = jnp.reshape(jnp.broadcast_to(z[:, None, :], (bq, nb, cg)),
                     (bq * nb, cg))
    h = g - ze + b_ref[...]
    h_ref[...] = h
    me = me_ref[...]
    hm = h * me
    ssum_ref[...] += jnp.sum(hm, axis=0, keepdims=True)
    ssq_ref[...] += jnp.sum(hm * h, axis=0, keepdims=True)
    cnt_ref[...] += jnp.sum(me)


def _edge_mid_body(h_ref, me_ref, ssum_ref, ssq_ref, cnt_ref, w_ref, b_ref,
                   ga_ref, be_ref, h2_ref, ssum2_ref, ssq2_ref):
    i = pl.program_id(0)

    @pl.when(i == 0)
    def _init():
        ssum2_ref[...] = jnp.zeros_like(ssum2_ref)
        ssq2_ref[...] = jnp.zeros_like(ssq2_ref)

    cnt = jnp.maximum(cnt_ref[0, 0], 1.0)
    mean = ssum_ref[...] / cnt
    var = ssq_ref[...] / cnt - mean * mean
    h = h_ref[...]
    hn = (h - mean) * jax.lax.rsqrt(var + 1e-5) * ga_ref[...] + be_ref[...]
    hn = jnp.maximum(hn, 0.0)
    h2 = jnp.dot(hn, w_ref[...], preferred_element_type=jnp.float32, precision=lax.Precision.HIGHEST) + b_ref[...]
    h2_ref[...] = h2
    me = me_ref[...]
    hm = h2 * me
    ssum2_ref[...] += jnp.sum(hm, axis=0, keepdims=True)
    ssq2_ref[...] += jnp.sum(hm * h2, axis=0, keepdims=True)


def _edge_last_body(h_ref, me_ref, ssum_ref, ssq_ref, cnt_ref, w_ref, b_ref,
                    ga_ref, be_ref, x_ref, *, bq, nb, cout):
    cnt = jnp.maximum(cnt_ref[0, 0], 1.0)
    mean = ssum_ref[...] / cnt
    var = ssq_ref[...] / cnt - mean * mean
    h = h_ref[...]
    hn = (h - mean) * jax.lax.rsqrt(var + 1e-5) * ga_ref[...] + be_ref[...]
    hn = jnp.maximum(hn, 0.0)
    h3 = jnp.dot(hn, w_ref[...], preferred_element_type=jnp.float32, precision=lax.Precision.HIGHEST) + b_ref[...]
    m3 = jnp.reshape(me_ref[...], (bq, nb, 1))
    h3 = jnp.reshape(h3, (bq, nb, cout))
    fmin = jnp.finfo(jnp.float32).min
    x_ref[...] = jnp.max(jnp.where(m3 > 0, h3, fmin), axis=1)


def _edge_mlp(g, z, maskE, layers, mpad, bq):
    nb = _MAX_NB
    E = mpad * nb
    c0 = z.shape[1]
    c1 = layers[1]["W"].shape[1]
    c2 = layers[2]["W"].shape[1]
    nblk = mpad // bq
    gw = g.shape[1]
    row = lambda v: v.reshape(1, -1)

    h1, ssum1, ssq1, cnt = pl.pallas_call(
        functools.partial(_edge1_body, bq=bq, nb=nb, cg=c0),
        grid=(nblk,),
        in_specs=[
            pl.BlockSpec((bq * nb, gw), lambda i: (i, 0)),
            pl.BlockSpec((bq, c0), lambda i: (i, 0)),
            pl.BlockSpec((bq * nb, 1), lambda i: (i, 0)),
            pl.BlockSpec((1, c0), lambda i: (0, 0)),
        ],
        out_specs=[
            pl.BlockSpec((bq * nb, c0), lambda i: (i, 0)),
            pl.BlockSpec((1, c0), lambda i: (0, 0)),
            pl.BlockSpec((1, c0), lambda i: (0, 0)),
            pl.BlockSpec((1, 128), lambda i: (0, 0)),
        ],
        out_shape=[
            jax.ShapeDtypeStruct((E, c0), jnp.float32),
            jax.ShapeDtypeStruct((1, c0), jnp.float32),
            jax.ShapeDtypeStruct((1, c0), jnp.float32),
            jax.ShapeDtypeStruct((1, 128), jnp.float32),
        ],
    )(g, z, maskE, row(layers[0]["b"]))

    h2, ssum2, ssq2 = pl.pallas_call(
        _edge_mid_body,
        grid=(nblk,),
        in_specs=[
            pl.BlockSpec((bq * nb, c0), lambda i: (i, 0)),
            pl.BlockSpec((bq * nb, 1), lambda i: (i, 0)),
            pl.BlockSpec((1, c0), lambda i: (0, 0)),
            pl.BlockSpec((1, c0), lambda i: (0, 0)),
            pl.BlockSpec((1, 128), lambda i: (0, 0)),
            pl.BlockSpec((c0, c1), lambda i: (0, 0)),
            pl.BlockSpec((1, c1), lambda i: (0, 0)),
            pl.BlockSpec((1, c0), lambda i: (0, 0)),
            pl.BlockSpec((1, c0), lambda i: (0, 0)),
        ],
        out_specs=[
            pl.BlockSpec((bq * nb, c1), lambda i: (i, 0)),
            pl.BlockSpec((1, c1), lambda i: (0, 0)),
            pl.BlockSpec((1, c1), lambda i: (0, 0)),
        ],
        out_shape=[
            jax.ShapeDtypeStruct((E, c1), jnp.float32),
            jax.ShapeDtypeStruct((1, c1), jnp.float32),
            jax.ShapeDtypeStruct((1, c1), jnp.float32),
        ],
    )(h1, maskE, ssum1, ssq1, cnt, layers[1]["W"], row(layers[1]["b"]),
      row(layers[0]["g"]), row(layers[0]["be"]))

    x = pl.pallas_call(
        functools.partial(_edge_last_body, bq=bq, nb=nb, cout=c2),
        grid=(nblk,),
        in_specs=[
            pl.BlockSpec((bq * nb, c1), lambda i: (i, 0)),
            pl.BlockSpec((bq * nb, 1), lambda i: (i, 0)),
            pl.BlockSpec((1, c1), lambda i: (0, 0)),
            pl.BlockSpec((1, c1), lambda i: (0, 0)),
            pl.BlockSpec((1, 128), lambda i: (0, 0)),
            pl.BlockSpec((c1, c2), lambda i: (0, 0)),
            pl.BlockSpec((1, c2), lambda i: (0, 0)),
            pl.BlockSpec((1, c1), lambda i: (0, 0)),
            pl.BlockSpec((1, c1), lambda i: (0, 0)),
        ],
        out_specs=pl.BlockSpec((bq, c2), lambda i: (i, 0)),
        out_shape=jax.ShapeDtypeStruct((mpad, c2), jnp.float32),
    )(h2, maskE, ssum2, ssq2, cnt, layers[2]["W"], row(layers[2]["b"]),
      row(layers[1]["g"]), row(layers[1]["be"]))
    return x


def _global_body(h_ref, vm_ref, bm_ref,
                 w0_ref, b0_ref, g0_ref, e0_ref,
                 w1_ref, b1_ref, g1_ref, e1_ref,
                 w2_ref, b2_ref,
                 hw0_ref, hb0_ref, hw1_ref, hb1_ref, hw2_ref, hb2_ref,
                 o_ref):
    vm = vm_ref[...]
    h = h_ref[...] * vm
    cnt = jnp.maximum(jnp.sum(vm), 1.0)

    def bn_layer(h, w, b, ga, be):
        h = jnp.dot(h, w, preferred_element_type=jnp.float32, precision=lax.Precision.HIGHEST) + b
        hm = h * vm
        mean = jnp.sum(hm, axis=0, keepdims=True) / cnt
        d = h - mean
        var = jnp.sum(d * d * vm, axis=0, keepdims=True) / cnt
        h = d * jax.lax.rsqrt(var + 1e-5) * ga + be
        return jnp.maximum(h, 0.0)

    h = bn_layer(h, w0_ref[...], b0_ref[...], g0_ref[...], e0_ref[...])
    h = bn_layer(h, w1_ref[...], b1_ref[...], g1_ref[...], e1_ref[...])
    h = jnp.dot(h, w2_ref[...], preferred_element_type=jnp.float32, precision=lax.Precision.HIGHEST) + b2_ref[...]

    fmin = jnp.finfo(jnp.float32).min
    bm = bm_ref[...]
    rows = []
    for gidx in range(_B):
        col = bm[:, gidx:gidx + 1]
        rows.append(jnp.max(jnp.where(col > 0, h, fmin), axis=0,
                            keepdims=True))
    g = jnp.concatenate(rows, axis=0)

    g = jnp.maximum(jnp.dot(g, hw0_ref[...],
                            preferred_element_type=jnp.float32, precision=lax.Precision.HIGHEST) + hb0_ref[...], 0.0)
    g = jnp.maximum(jnp.dot(g, hw1_ref[...],
                            preferred_element_type=jnp.float32, precision=lax.Precision.HIGHEST) + hb1_ref[...], 0.0)
    out = jnp.dot(g, hw2_ref[...],
                  preferred_element_type=jnp.float32, precision=lax.Precision.HIGHEST) + hb2_ref[...]
    mx = jnp.max(out, axis=1, keepdims=True)
    ex = jnp.exp(out - mx)
    o_ref[...] = out - mx - jnp.log(jnp.sum(ex, axis=1, keepdims=True))


def _global_head(hcat, validf, bmaskT, gparams, hparams):
    row = lambda v: v.reshape(1, -1)
    args = [hcat, validf, bmaskT,
            gparams[0]["W"], row(gparams[0]["b"]), row(gparams[0]["g"]),
            row(gparams[0]["be"]),
            gparams[1]["W"], row(gparams[1]["b"]), row(gparams[1]["g"]),
            row(gparams[1]["be"]),
            gparams[2]["W"], row(gparams[2]["b"]),
            hparams[0]["W"], row(hparams[0]["b"]),
            hparams[1]["W"], row(hparams[1]["b"]),
            hparams[2]["W"], row(hparams[2]["b"])]
    return pl.pallas_call(
        _global_body,
        out_shape=jax.ShapeDtypeStruct((_B, 10), jnp.float32),
    )(*args)


def _masked_bn(h, mask, g, be, eps=1e-5):
    m = mask[..., None].astype(h.dtype)
    cnt = jnp.maximum(jnp.sum(m), 1.0)
    mean = jnp.sum(h * m, axis=(0, 1)) / cnt
    var = jnp.sum(((h - mean) ** 2) * m, axis=(0, 1)) / cnt
    return (h - mean) / jnp.sqrt(var + eps) * g + be


def _mlp_edge(h, layers, mask):
    for i, lp in enumerate(layers):
        h = h @ lp["W"] + lp["b"]
        if i < len(layers) - 1:
            h = _masked_bn(h, mask, lp["g"], lp["be"])
            h = jax.nn.relu(h)
    return h


def _mlp_node(h, layers, use_bn):
    for i, lp in enumerate(layers):
        h = h @ lp["W"] + lp["b"]
        if i < len(layers) - 1:
            if use_bn:
                mean = jnp.mean(h, axis=0)
                var = jnp.mean((h - mean) ** 2, axis=0)
                h = (h - mean) / jnp.sqrt(var + 1e-5) * lp["g"] + lp["be"]
            h = jax.nn.relu(h)
    return h


def _mlp_node_masked(h, layers, valid, eps=1e-5):
    m = valid[:, None].astype(h.dtype)
    cnt = jnp.maximum(jnp.sum(m), 1.0)
    for i, lp in enumerate(layers):
        h = h @ lp["W"] + lp["b"]
        if i < len(layers) - 1:
            mean = jnp.sum(h * m, axis=0) / cnt
            var = jnp.sum(((h - mean) ** 2) * m, axis=0) / cnt
            h = (h - mean) / jnp.sqrt(var + eps) * lp["g"] + lp["be"]
            h = jax.nn.relu(h)
    return h


def _masked_max(h, mask):
    return jnp.max(jnp.where(mask[..., None], h, jnp.finfo(h.dtype).min), axis=1)


def kernel(pos, batch, params):
    n = pos.shape[0]
    counts = jnp.bincount(batch, length=_B)
    starts = jnp.concatenate([jnp.zeros((1,), counts.dtype), jnp.cumsum(counts)[:-1]])
    cap1 = n // 2 + _B
    ppad1, mcap1 = _N, 4096
    idx1, bid1, v1, mvec1, offs1, jsel1 = _fps_level(
        pos, starts, counts, _RATIO1, cap1, ppad1, mcap1)
    nbr1, m1 = _radius_sc(pos, starts, counts, idx1, bid1, v1,
                          _r2_f32(_R1), _MAX_NB, 4608, _N)
    pos1 = pos[idx1]
    starts1 = offs1[:-1]
    cap2 = cap1 // 4 + _B
    ppad2, mcap2 = 4224, 1024
    idx2, bid2, v2, mvec2, offs2, jsel2 = _fps_level(
        pos1, starts1, mvec1, _RATIO2, cap2, ppad2, mcap2)
    nbr2, m2 = _radius_sc(pos1, starts1, mvec1, idx2, bid2, v2,
                          _r2_f32(_R2), _MAX_NB, 1536, 4224)
    batch2 = jnp.where(v2, bid2, _B)
    mpad1, mpad2 = 4608, 1536

    def padk(x, k):
        return jnp.pad(x, ((0, 0), (0, k - x.shape[1])))

    w1 = params["local1"][0]["W"]
    ap = _mm(padk(jnp.concatenate([pos, pos1], axis=0), 128), padk(w1.T, 128).T)
    a1 = ap[:n]
    z1 = jnp.pad(ap[n:], ((0, mpad1 - cap1), (0, 0)))
    nbrf1 = jnp.pad(nbr1, ((0, mpad1 - cap1), (0, 0))).reshape(-1).astype(jnp.int32)
    me1 = jnp.pad(m1, ((0, mpad1 - cap1), (0, 0))).reshape(-1, 1).astype(jnp.float32)
    g1 = _sc_gather(padk(a1, 128), nbrf1)
    x1 = _edge_mlp(g1, z1, me1, params["local1"], mpad1, 128)

    pos2 = pos1[idx2]
    w2 = params["local2"][0]["W"]
    y2 = _mm(padk(jnp.concatenate([x1[:cap1], pos1], axis=1), 256),
             padk(w2.T, 256).T)
    z2 = _mm(padk(jnp.pad(pos2, ((0, mpad2 - cap2), (0, 0))), 128),
             padk(w2[128:].T, 128).T)
    nbrf2 = jnp.pad(nbr2, ((0, mpad2 - cap2), (0, 0))).reshape(-1).astype(jnp.int32)
    me2 = jnp.pad(m2, ((0, mpad2 - cap2), (0, 0))).reshape(-1, 1).astype(jnp.float32)
    g2 = _sc_gather(y2, nbrf2)
    x2 = _edge_mlp(g2, z2, me2, params["local2"], mpad2, 128)

    hcat = jnp.concatenate(
        [x2, jnp.pad(pos2, ((0, mpad2 - cap2), (0, 0)))], axis=1)
    validf = jnp.pad(v2.astype(jnp.float32), (0, mpad2 - cap2)).reshape(-1, 1)
    bmaskT = (jnp.pad(batch2, (0, mpad2 - cap2), constant_values=_B)[:, None]
              == jnp.arange(_B)[None, :]).astype(jnp.float32)
    return _global_head(hcat, validf, bmaskT, params["global"], params["head"])
